# baseline (speedup 1.0000x reference)
import jax
import jax.numpy as jnp
from jax.experimental import pallas as pl
from jax.experimental.pallas import tpu as pltpu

B = 8192
SEQ = 1024
RB = 32
BB = RB * 128
NJ = 8
RPJ = RB // NJ
TC = 256
NP = SEQ // TC

_LOG2E = 1.4426950408889634


def _leaky(v):
    return jnp.where(v > 0, v, 0.2 * v)


def _cell(xf, xi, xg, xo, c):
    ef = jnp.exp2(xf)
    ei = jnp.exp2(xi)
    eg = jnp.exp2(xg)
    eo = jnp.exp2(xo)
    a1 = 1.0 + ef
    b1_ = 1.0 + ei
    c1 = 1.0 + eg
    g1 = 1.0 - eg
    bc = b1_ * c1
    c_new = (c * bc + g1 * a1) * (1.0 / (a1 * bc))
    ec = jnp.exp2(jnp.minimum(c_new * (-2.0 * _LOG2E), 100.0))
    h_new = (1.0 - ec) * (1.0 / ((1.0 + ec) * (1.0 + eo)))
    return h_new, c_new


def _lstm_kernel(wih_ref, whh_ref, bih_ref, bhh_ref,
                 x_ref, h0_ref, c0_ref,
                 w1_ref, b1_ref, w2_ref, b2_ref, w3_ref, b3_ref,
                 w4_ref, b4_ref, wout_ref, bout_ref,
                 out_ref, hbm_ref, s_ref, s2_ref, sema_ref, semb_ref):
    i = pl.program_id(0)
    j = pl.program_id(1)

    def _a_copies(p):
        for r in range(RB):
            yield pltpu.make_async_copy(
                s_ref.at[pl.ds(p * TC, TC), r, :],
                hbm_ref.at[i, p, r],
                sema_ref)

    def _b_copies(p):
        for r in range(RB):
            yield pltpu.make_async_copy(
                hbm_ref.at[i, p, r],
                s2_ref.at[r, pl.ds(p * TC, TC), :],
                semb_ref)

    @pl.when(j == 0)
    def _run_lstm():
        cd_tb = (((1,), (1,)), ((), ()))
        cd_nn = (((1,), (0,)), ((), ()))
        a = _leaky(jax.lax.dot_general(
            w1_ref[...], x_ref[...], cd_tb,
            preferred_element_type=jnp.float32) + b1_ref[...])
        a = _leaky(jax.lax.dot_general(
            w2_ref[...], a, cd_nn,
            preferred_element_type=jnp.float32) + b2_ref[...])
        a = _leaky(jax.lax.dot_general(
            w3_ref[...], a, cd_nn,
            preferred_element_type=jnp.float32) + b3_ref[...])
        zt = _leaky(jax.lax.dot_general(
            w4_ref[...], a, cd_nn,
            preferred_element_type=jnp.float32) + b4_ref[...])
        z = jnp.concatenate(
            [zt[:, r * 128:(r + 1) * 128] for r in range(RB)], axis=0)

        wii, wif, wig, wio = (wih_ref[k, 0] for k in range(4))
        whi, whf, whg, who = (whh_ref[k, 0] for k in range(4))
        bi, bf, bg, bo = (bih_ref[k] + bhh_ref[k] for k in range(4))

        nl = -_LOG2E
        h = h0_ref[...]
        c = c0_ref[...]
        clamp = lambda v: jnp.minimum(v, 100.0)
        h, c = _cell(
            clamp(z * (wif * nl) + h * (whf * nl) + bf * nl),
            clamp(z * (wii * nl) + h * (whi * nl) + bi * nl),
            clamp(z * (wig * 2 * nl) + h * (whg * 2 * nl) + bg * 2 * nl),
            clamp(z * (wio * nl) + h * (who * nl) + bo * nl),
            c)
        s_ref[pl.ds(0, 1)] = h[None]

        kf, cf = (wif + whf) * nl, bf * nl
        ki, ci = (wii + whi) * nl, bi * nl
        kg, cg = (wig + whg) * 2 * nl, bg * 2 * nl
        ko, co = (wio + who) * nl, bo * nl

        h, c = _cell(h * kf + cf, h * ki + ci, h * kg + cg, h * ko + co, c)
        s_ref[pl.ds(1, 1)] = h[None]

        def make_body(base):
            def body(k, carry):
                h, c, kf, cf, ki, ci, kg, cg, ko, co = carry
                t = base + 2 * k
                h, c = _cell(h * kf + cf, h * ki + ci,
                             h * kg + cg, h * ko + co, c)
                s_ref[pl.ds(t, 1)] = h[None]
                h, c = _cell(h * kf + cf, h * ki + ci,
                             h * kg + cg, h * ko + co, c)
                s_ref[pl.ds(t + 1, 1)] = h[None]
                return (h, c, kf, cf, ki, ci, kg, cg, ko, co)
            return body

        consts = (kf, cf, ki, ci, kg, cg, ko, co)
        for p in range(NP):
            n2 = (TC - 2) // 2 if p == 0 else TC // 2
            carry = jax.lax.fori_loop(0, n2, make_body(2 if p == 0 else p * TC),
                                      (h, c) + consts)
            h, c = carry[0], carry[1]
            for cp in _a_copies(p):
                cp.start()
            if p >= 1:
                for cp in _a_copies(p - 1):
                    cp.wait()
                for cp in _b_copies(p - 1):
                    cp.start()
        for cp in _a_copies(NP - 1):
            cp.wait()
        for cp in _b_copies(NP - 1):
            cp.start()
        for p in range(NP):
            for cp in _b_copies(p):
                cp.wait()

    wout_blk = wout_ref[...]
    bias = bout_ref[...]
    for q in range(RPJ):
        r = j * RPJ + q
        acc = jax.lax.dot_general(s2_ref[r], wout_blk,
                                  (((0,), (1,)), ((), ())),
                                  preferred_element_type=jnp.float32)
        out_ref[q * 128:(q + 1) * 128, :] = acc + bias


def kernel(x, h0, c0, W1, b1, W2, b2, W3, b3, W4, b4,
           W_ih, b_ih, W_hh, b_hh, Wout, bout):
    h02 = h0.reshape(B // 128, 128)
    c02 = c0.reshape(B // 128, 128)

    smem = lambda: pl.BlockSpec(memory_space=pltpu.SMEM)
    full = lambda shape: pl.BlockSpec(shape, lambda i, j: (0,) * len(shape))

    out, _ = pl.pallas_call(
        _lstm_kernel,
        grid=(2, NJ),
        in_specs=[
            smem(), smem(), smem(), smem(),
            pl.BlockSpec((BB, 8), lambda i, j: (i, 0)),
            pl.BlockSpec((RB, 128), lambda i, j: (i, 0)),
            pl.BlockSpec((RB, 128), lambda i, j: (i, 0)),
            full((64, 8)), full((64, 1)),
            full((128, 64)), full((128, 1)),
            full((4, 128)), full((4, 1)),
            full((1, 4)), full((1, 1)),
            full((SEQ, SEQ)), full((1, SEQ)),
        ],
        out_specs=[
            pl.BlockSpec((BB // NJ, SEQ), lambda i, j: (i * NJ + j, 0)),
            pl.BlockSpec(memory_space=pl.ANY),
        ],
        out_shape=[
            jax.ShapeDtypeStruct((B, SEQ), jnp.float32),
            jax.ShapeDtypeStruct((2, NP, RB, TC, 128), jnp.float32),
        ],
        scratch_shapes=[pltpu.VMEM((SEQ, RB, 128), jnp.float32),
                        pltpu.VMEM((RB, SEQ, 128), jnp.float32),
                        pltpu.SemaphoreType.DMA,
                        pltpu.SemaphoreType.DMA],
        compiler_params=pltpu.CompilerParams(
            dimension_semantics=("arbitrary", "arbitrary"),
            vmem_limit_bytes=60 * 1024 * 1024),
    )(W_ih, W_hh, b_ih, b_hh,
      x, h02, c02,
      W1, b1[:, None], W2, b2[:, None], W3, b3[:, None], W4, b4[:, None],
      Wout, bout[None])
    return out


# speedup vs baseline: 1.0077x; 1.0077x over previous
import jax
import jax.numpy as jnp
from jax.experimental import pallas as pl
from jax.experimental.pallas import tpu as pltpu

B = 8192
SEQ = 1024
RB = 32
BB = RB * 128
NJ = 8
RPJ = RB // NJ
TC = 256
NP = SEQ // TC

_LOG2E = 1.4426950408889634


def _leaky(v):
    return jnp.where(v > 0, v, 0.2 * v)


def _cell(xf, xi, xg, xo, c):
    ef = jnp.exp2(xf)
    ei = jnp.exp2(xi)
    eg = jnp.exp2(xg)
    eo = jnp.exp2(xo)
    a1 = 1.0 + ef
    b1_ = 1.0 + ei
    c1 = 1.0 + eg
    g1 = 1.0 - eg
    bc = b1_ * c1
    c_new = (c * bc + g1 * a1) * (1.0 / (a1 * bc))
    ec = jnp.exp2(jnp.minimum(c_new * (-2.0 * _LOG2E), 100.0))
    h_new = (1.0 - ec) * (1.0 / ((1.0 + ec) * (1.0 + eo)))
    return h_new, c_new


def _lstm_kernel(wih_ref, whh_ref, bih_ref, bhh_ref,
                 x_ref, h0_ref, c0_ref,
                 w1_ref, b1_ref, w2_ref, b2_ref, w3_ref, b3_ref,
                 w4_ref, b4_ref, wout_ref, bout_ref,
                 out_ref, hbm_ref, s_ref, s2_ref, sema_ref, semb_ref):
    i = pl.program_id(0)
    j = pl.program_id(1)

    def _a_copies(p):
        for r in range(RB):
            yield pltpu.make_async_copy(
                s_ref.at[pl.ds(p * TC, TC), r, :],
                hbm_ref.at[i, p, r],
                sema_ref)

    def _b_copies(p):
        for r in range(RB):
            yield pltpu.make_async_copy(
                hbm_ref.at[i, p, r],
                s2_ref.at[r, pl.ds(p * TC, TC), :],
                semb_ref)

    @pl.when(j == 0)
    def _run_lstm():
        cd_tb = (((1,), (1,)), ((), ()))
        cd_nn = (((1,), (0,)), ((), ()))
        a = _leaky(jax.lax.dot_general(
            w1_ref[...], x_ref[...], cd_tb,
            preferred_element_type=jnp.float32) + b1_ref[...])
        a = _leaky(jax.lax.dot_general(
            w2_ref[...], a, cd_nn,
            preferred_element_type=jnp.float32) + b2_ref[...])
        a = _leaky(jax.lax.dot_general(
            w3_ref[...], a, cd_nn,
            preferred_element_type=jnp.float32) + b3_ref[...])
        zt = _leaky(jax.lax.dot_general(
            w4_ref[...], a, cd_nn,
            preferred_element_type=jnp.float32) + b4_ref[...])
        z = jnp.concatenate(
            [zt[:, r * 128:(r + 1) * 128] for r in range(RB)], axis=0)

        wii, wif, wig, wio = (wih_ref[k, 0] for k in range(4))
        whi, whf, whg, who = (whh_ref[k, 0] for k in range(4))
        bi, bf, bg, bo = (bih_ref[k] + bhh_ref[k] for k in range(4))

        nl = -_LOG2E
        h = h0_ref[...]
        c = c0_ref[...]
        clamp = lambda v: jnp.minimum(v, 100.0)
        h, c = _cell(
            clamp(z * (wif * nl) + h * (whf * nl) + bf * nl),
            clamp(z * (wii * nl) + h * (whi * nl) + bi * nl),
            clamp(z * (wig * 2 * nl) + h * (whg * 2 * nl) + bg * 2 * nl),
            clamp(z * (wio * nl) + h * (who * nl) + bo * nl),
            c)
        s_ref[pl.ds(0, 1)] = h[None]

        kf, cf = (wif + whf) * nl, bf * nl
        ki, ci = (wii + whi) * nl, bi * nl
        kg, cg = (wig + whg) * 2 * nl, bg * 2 * nl
        ko, co = (wio + who) * nl, bo * nl

        h, c = _cell(h * kf + cf, h * ki + ci, h * kg + cg, h * ko + co, c)
        s_ref[pl.ds(1, 1)] = h[None]

        def make_body(base, unroll):
            def body(k, carry):
                h, c, kf, cf, ki, ci, kg, cg, ko, co = carry
                t = base + unroll * k
                for u in range(unroll):
                    h, c = _cell(h * kf + cf, h * ki + ci,
                                 h * kg + cg, h * ko + co, c)
                    s_ref[pl.ds(t + u, 1)] = h[None]
                return (h, c, kf, cf, ki, ci, kg, cg, ko, co)
            return body

        consts = (kf, cf, ki, ci, kg, cg, ko, co)
        for p in range(NP):
            un = 2 if p == 0 else 4
            n2 = (TC - 2) // un if p == 0 else TC // un
            carry = jax.lax.fori_loop(0, n2,
                                      make_body(2 if p == 0 else p * TC, un),
                                      (h, c) + consts)
            h, c = carry[0], carry[1]
            for cp in _a_copies(p):
                cp.start()
            if p >= 1:
                for cp in _a_copies(p - 1):
                    cp.wait()
                for cp in _b_copies(p - 1):
                    cp.start()
        for cp in _a_copies(NP - 1):
            cp.wait()
        for cp in _b_copies(NP - 1):
            cp.start()
        for p in range(NP):
            for cp in _b_copies(p):
                cp.wait()

    wout_blk = wout_ref[...]
    bias = bout_ref[...]
    for q in range(RPJ):
        r = j * RPJ + q
        acc = jax.lax.dot_general(s2_ref[r], wout_blk,
                                  (((0,), (1,)), ((), ())),
                                  preferred_element_type=jnp.float32)
        out_ref[q * 128:(q + 1) * 128, :] = acc + bias


def kernel(x, h0, c0, W1, b1, W2, b2, W3, b3, W4, b4,
           W_ih, b_ih, W_hh, b_hh, Wout, bout):
    h02 = h0.reshape(B // 128, 128)
    c02 = c0.reshape(B // 128, 128)

    smem = lambda: pl.BlockSpec(memory_space=pltpu.SMEM)
    full = lambda shape: pl.BlockSpec(shape, lambda i, j: (0,) * len(shape))

    out, _ = pl.pallas_call(
        _lstm_kernel,
        grid=(2, NJ),
        in_specs=[
            smem(), smem(), smem(), smem(),
            pl.BlockSpec((BB, 8), lambda i, j: (i, 0)),
            pl.BlockSpec((RB, 128), lambda i, j: (i, 0)),
            pl.BlockSpec((RB, 128), lambda i, j: (i, 0)),
            full((64, 8)), full((64, 1)),
            full((128, 64)), full((128, 1)),
            full((4, 128)), full((4, 1)),
            full((1, 4)), full((1, 1)),
            full((SEQ, SEQ)), full((1, SEQ)),
        ],
        out_specs=[
            pl.BlockSpec((BB // NJ, SEQ), lambda i, j: (i * NJ + j, 0)),
            pl.BlockSpec(memory_space=pl.ANY),
        ],
        out_shape=[
            jax.ShapeDtypeStruct((B, SEQ), jnp.float32),
            jax.ShapeDtypeStruct((2, NP, RB, TC, 128), jnp.float32),
        ],
        scratch_shapes=[pltpu.VMEM((SEQ, RB, 128), jnp.float32),
                        pltpu.VMEM((RB, SEQ, 128), jnp.float32),
                        pltpu.SemaphoreType.DMA,
                        pltpu.SemaphoreType.DMA],
        compiler_params=pltpu.CompilerParams(
            dimension_semantics=("arbitrary", "arbitrary"),
            vmem_limit_bytes=60 * 1024 * 1024),
    )(W_ih, W_hh, b_ih, b_hh,
      x, h02, c02,
      W1, b1[:, None], W2, b2[:, None], W3, b3[:, None], W4, b4[:, None],
      Wout, bout[None])
    return out
